# Initial kernel scaffold; baseline (speedup 1.0000x reference)
import jax
import jax.numpy as jnp
from jax.experimental import pallas as pl


def kernel(inputs, targets, features):
    raise NotImplementedError("write your pallas kernel here")


# speedup vs baseline: 4.3566x; 4.3566x over previous
import functools

import jax
import jax.numpy as jnp
from jax import lax
from jax.experimental import pallas as pl
from jax.experimental.pallas import tpu as pltpu
from jax.experimental.pallas import tpu_sc as plsc

B = 4096
D = 256
K = 8192
TEMP = 0.05
BR = 256
COLT = 2048
NBLK = B // BR


def _sc_gather(targets, feats):
    info = plsc.get_sparse_core_info()
    nw = info.num_cores * info.num_subcores
    bpw = B // nw
    mesh = plsc.VectorSubcoreMesh(core_axis_name="c", subcore_axis_name="s")

    @functools.partial(
        pl.kernel, mesh=mesh,
        out_type=(jax.ShapeDtypeStruct((B, D), jnp.float32),
                  jax.ShapeDtypeStruct((B, D), jnp.float32)),
        scratch_types=[
            pltpu.VMEM((bpw,), jnp.int32),
            pltpu.VMEM((bpw,), jnp.int32),
            pltpu.VMEM((bpw, D), jnp.float32),
            pltpu.VMEM((bpw, D), jnp.float32),
            pltpu.SemaphoreType.DMA,
        ],
    )
    def k(tgt_hbm, feats_hbm, outm_hbm, outh_hbm, idx_v, idx2_v,
          rows_m, rows_h, sem):
        wid = lax.axis_index("s") * info.num_cores + lax.axis_index("c")
        base = wid * bpw
        pltpu.sync_copy(tgt_hbm.at[pl.ds(base, bpw)], idx_v)
        for j in range(bpw // 16):
            sl = pl.ds(j * 16, 16)
            idx2_v[sl] = idx_v[sl] + K
        pltpu.async_copy(feats_hbm.at[idx_v], rows_m, sem).wait()
        pltpu.async_copy(feats_hbm.at[idx2_v], rows_h, sem).wait()
        pltpu.sync_copy(rows_m, outm_hbm.at[pl.ds(base, bpw)])
        pltpu.sync_copy(rows_h, outh_hbm.at[pl.ds(base, bpw)])

    return k(targets, feats)


def _tc_body(x_ref, gm_ref, gh_ref, feats_ref, out_ref):
    i = pl.program_id(0)
    x = x_ref[...]
    norm = jnp.sqrt(jnp.sum(x * x, axis=1, keepdims=True))
    xn = x / jnp.maximum(norm, 1e-12)
    t_m = jnp.sum(xn * gm_ref[...], axis=1) * (1.0 / TEMP)
    t_h = jnp.sum(xn * gh_ref[...], axis=1) * (1.0 / TEMP)
    acc_m = jnp.zeros((BR,), jnp.float32)
    acc_h = jnp.zeros((BR,), jnp.float32)
    for c in range(K // COLT):
        f_m = feats_ref[pl.ds(c * COLT, COLT), :]
        l_m = lax.dot_general(xn, f_m, (((1,), (1,)), ((), ())),
                              preferred_element_type=jnp.float32)
        acc_m = acc_m + jnp.sum(jnp.exp(l_m * (1.0 / TEMP) - 20.0), axis=1)
        f_h = feats_ref[pl.ds(K + c * COLT, COLT), :]
        l_h = lax.dot_general(xn, f_h, (((1,), (1,)), ((), ())),
                              preferred_element_type=jnp.float32)
        acc_h = acc_h + jnp.sum(jnp.exp(l_h * (1.0 / TEMP) - 20.0), axis=1)
    lse_m = jnp.log(acc_m) + 20.0
    lse_h = jnp.log(acc_h) + 20.0
    block = jnp.sum((lse_m - t_m) + (lse_h - t_h))

    @pl.when(i == 0)
    def _init():
        out_ref[0, 0] = 0.0

    out_ref[0, 0] += block

    @pl.when(i == NBLK - 1)
    def _fin():
        out_ref[0, 0] = out_ref[0, 0] * (0.5 / B)


def _tc_call(x, g_m, g_h, feats, interpret=False):
    return pl.pallas_call(
        _tc_body,
        grid=(NBLK,),
        in_specs=[
            pl.BlockSpec((BR, D), lambda i: (i, 0)),
            pl.BlockSpec((BR, D), lambda i: (i, 0)),
            pl.BlockSpec((BR, D), lambda i: (i, 0)),
            pl.BlockSpec((2 * K, D), lambda i: (0, 0)),
        ],
        out_specs=pl.BlockSpec((1, 1), lambda i: (0, 0),
                               memory_space=pltpu.SMEM),
        out_shape=jax.ShapeDtypeStruct((1, 1), jnp.float32),
        interpret=interpret,
    )(x, g_m, g_h, feats)


def kernel(inputs, targets, features):
    tgt = targets.astype(jnp.int32)
    g_m, g_h = _sc_gather(tgt, features)
    out = _tc_call(inputs, g_m, g_h, features)
    return out[0, 0]


# speedup vs baseline: 5.5796x; 1.2807x over previous
import functools

import jax
import jax.numpy as jnp
from jax import lax
from jax.experimental import pallas as pl
from jax.experimental.pallas import tpu as pltpu
from jax.experimental.pallas import tpu_sc as plsc

B = 4096
D = 256
K = 8192
TEMP = 0.05
BR = 256
COLT = 2048
NBLK = B // BR


def _sc_gather(targets, feats):
    info = plsc.get_sparse_core_info()
    nw = info.num_cores * info.num_subcores
    bpw = B // nw
    mesh = plsc.VectorSubcoreMesh(core_axis_name="c", subcore_axis_name="s")

    @functools.partial(
        pl.kernel, mesh=mesh,
        out_type=(jax.ShapeDtypeStruct((B, D), jnp.float32),
                  jax.ShapeDtypeStruct((B, D), jnp.float32)),
        scratch_types=[
            pltpu.VMEM((bpw,), jnp.int32),
            pltpu.VMEM((bpw,), jnp.int32),
            pltpu.VMEM((bpw, D), jnp.float32),
            pltpu.VMEM((bpw, D), jnp.float32),
            pltpu.SemaphoreType.DMA,
        ],
    )
    def k(tgt_hbm, feats_hbm, outm_hbm, outh_hbm, idx_v, idx2_v,
          rows_m, rows_h, sem):
        wid = lax.axis_index("s") * info.num_cores + lax.axis_index("c")
        base = wid * bpw
        pltpu.sync_copy(tgt_hbm.at[pl.ds(base, bpw)], idx_v)
        for j in range(bpw // 16):
            sl = pl.ds(j * 16, 16)
            idx2_v[sl] = idx_v[sl] + K
        pltpu.async_copy(feats_hbm.at[idx_v], rows_m, sem).wait()
        pltpu.async_copy(feats_hbm.at[idx2_v], rows_h, sem).wait()
        pltpu.sync_copy(rows_m, outm_hbm.at[pl.ds(base, bpw)])
        pltpu.sync_copy(rows_h, outh_hbm.at[pl.ds(base, bpw)])

    return k(targets, feats)


def _tc_body(x_ref, gm_ref, gh_ref, feats_ref, out_ref):
    i = pl.program_id(0)
    x = x_ref[...]
    norm = jnp.sqrt(jnp.sum(x * x, axis=1, keepdims=True))
    xn = x / jnp.maximum(norm, 1e-12)
    t_m = jnp.sum(xn * gm_ref[...], axis=1) * (1.0 / TEMP)
    t_h = jnp.sum(xn * gh_ref[...], axis=1) * (1.0 / TEMP)
    xnb = (xn * (1.0 / TEMP)).astype(jnp.bfloat16)
    acc_m = jnp.zeros((BR,), jnp.float32)
    acc_h = jnp.zeros((BR,), jnp.float32)
    for c in range(K // COLT):
        f_m = feats_ref[pl.ds(c * COLT, COLT), :]
        l_m = lax.dot_general(xnb, f_m, (((1,), (1,)), ((), ())),
                              preferred_element_type=jnp.float32)
        acc_m = acc_m + jnp.sum(jnp.exp(l_m), axis=1)
        f_h = feats_ref[pl.ds(K + c * COLT, COLT), :]
        l_h = lax.dot_general(xnb, f_h, (((1,), (1,)), ((), ())),
                              preferred_element_type=jnp.float32)
        acc_h = acc_h + jnp.sum(jnp.exp(l_h), axis=1)
    lse_m = jnp.log(acc_m)
    lse_h = jnp.log(acc_h)
    block = jnp.sum((lse_m - t_m) + (lse_h - t_h))

    @pl.when(i == 0)
    def _init():
        out_ref[0, 0] = 0.0

    out_ref[0, 0] += block

    @pl.when(i == NBLK - 1)
    def _fin():
        out_ref[0, 0] = out_ref[0, 0] * (0.5 / B)


def _tc_call(x, g_m, g_h, feats_bf16, interpret=False):
    return pl.pallas_call(
        _tc_body,
        grid=(NBLK,),
        in_specs=[
            pl.BlockSpec((BR, D), lambda i: (i, 0)),
            pl.BlockSpec((BR, D), lambda i: (i, 0)),
            pl.BlockSpec((BR, D), lambda i: (i, 0)),
            pl.BlockSpec((2 * K, D), lambda i: (0, 0)),
        ],
        out_specs=pl.BlockSpec((1, 1), lambda i: (0, 0),
                               memory_space=pltpu.SMEM),
        out_shape=jax.ShapeDtypeStruct((1, 1), jnp.float32),
        interpret=interpret,
    )(x, g_m, g_h, feats_bf16)


def kernel(inputs, targets, features):
    tgt = targets.astype(jnp.int32)
    g_m, g_h = _sc_gather(tgt, features)
    out = _tc_call(inputs, g_m, g_h, features.astype(jnp.bfloat16))
    return out[0, 0]


# speedup vs baseline: 5.9127x; 1.0597x over previous
import functools

import jax
import jax.numpy as jnp
from jax import lax
from jax.experimental import pallas as pl
from jax.experimental.pallas import tpu as pltpu
from jax.experimental.pallas import tpu_sc as plsc

B = 4096
D = 256
K = 8192
TEMP = 0.05
BR = 512
COLT = 2048
NBLK = B // BR


def _sc_gather(targets, feats):
    info = plsc.get_sparse_core_info()
    nw = info.num_cores * info.num_subcores
    bpw = B // nw
    mesh = plsc.VectorSubcoreMesh(core_axis_name="c", subcore_axis_name="s")

    @functools.partial(
        pl.kernel, mesh=mesh,
        out_type=(jax.ShapeDtypeStruct((B, D), jnp.float32),
                  jax.ShapeDtypeStruct((B, D), jnp.float32)),
        scratch_types=[
            pltpu.VMEM((bpw,), jnp.int32),
            pltpu.VMEM((bpw,), jnp.int32),
            pltpu.VMEM((bpw, D), jnp.float32),
            pltpu.VMEM((bpw, D), jnp.float32),
            pltpu.SemaphoreType.DMA,
        ],
    )
    def k(tgt_hbm, feats_hbm, outm_hbm, outh_hbm, idx_v, idx2_v,
          rows_m, rows_h, sem):
        wid = lax.axis_index("s") * info.num_cores + lax.axis_index("c")
        base = wid * bpw
        pltpu.sync_copy(tgt_hbm.at[pl.ds(base, bpw)], idx_v)
        for j in range(bpw // 16):
            sl = pl.ds(j * 16, 16)
            idx2_v[sl] = idx_v[sl] + K
        pltpu.async_copy(feats_hbm.at[idx_v], rows_m, sem).wait()
        pltpu.async_copy(feats_hbm.at[idx2_v], rows_h, sem).wait()
        pltpu.sync_copy(rows_m, outm_hbm.at[pl.ds(base, bpw)])
        pltpu.sync_copy(rows_h, outh_hbm.at[pl.ds(base, bpw)])

    return k(targets, feats)


def _tc_body(x_ref, gm_ref, gh_ref, feats_ref, out_ref, fb_ref):
    i = pl.program_id(0)

    @pl.when(i == 0)
    def _cast():
        fb_ref[...] = feats_ref[...].astype(jnp.bfloat16)

    x = x_ref[...]
    norm = jnp.sqrt(jnp.sum(x * x, axis=1, keepdims=True))
    xn = x / jnp.maximum(norm, 1e-12)
    t_m = jnp.sum(xn * gm_ref[...], axis=1) * (1.0 / TEMP)
    t_h = jnp.sum(xn * gh_ref[...], axis=1) * (1.0 / TEMP)
    xnb = (xn * (1.0 / TEMP)).astype(jnp.bfloat16)
    acc_m = jnp.zeros((BR,), jnp.float32)
    acc_h = jnp.zeros((BR,), jnp.float32)
    for c in range(K // COLT):
        f_m = fb_ref[pl.ds(c * COLT, COLT), :]
        l_m = lax.dot_general(xnb, f_m, (((1,), (1,)), ((), ())),
                              preferred_element_type=jnp.float32)
        acc_m = acc_m + jnp.sum(jnp.exp(l_m), axis=1)
        f_h = fb_ref[pl.ds(K + c * COLT, COLT), :]
        l_h = lax.dot_general(xnb, f_h, (((1,), (1,)), ((), ())),
                              preferred_element_type=jnp.float32)
        acc_h = acc_h + jnp.sum(jnp.exp(l_h), axis=1)
    lse_m = jnp.log(acc_m)
    lse_h = jnp.log(acc_h)
    block = jnp.sum((lse_m - t_m) + (lse_h - t_h))

    @pl.when(i == 0)
    def _init():
        out_ref[0, 0] = 0.0

    out_ref[0, 0] += block

    @pl.when(i == NBLK - 1)
    def _fin():
        out_ref[0, 0] = out_ref[0, 0] * (0.5 / B)


def _tc_call(x, g_m, g_h, feats, interpret=False):
    return pl.pallas_call(
        _tc_body,
        grid=(NBLK,),
        in_specs=[
            pl.BlockSpec((BR, D), lambda i: (i, 0)),
            pl.BlockSpec((BR, D), lambda i: (i, 0)),
            pl.BlockSpec((BR, D), lambda i: (i, 0)),
            pl.BlockSpec((2 * K, D), lambda i: (0, 0)),
        ],
        out_specs=pl.BlockSpec((1, 1), lambda i: (0, 0),
                               memory_space=pltpu.SMEM),
        out_shape=jax.ShapeDtypeStruct((1, 1), jnp.float32),
        scratch_shapes=[pltpu.VMEM((2 * K, D), jnp.bfloat16)],
        interpret=interpret,
    )(x, g_m, g_h, feats)


def kernel(inputs, targets, features):
    tgt = targets.astype(jnp.int32)
    g_m, g_h = _sc_gather(tgt, features)
    out = _tc_call(inputs, g_m, g_h, features)
    return out[0, 0]


# speedup vs baseline: 5.9396x; 1.0046x over previous
import functools

import jax
import jax.numpy as jnp
from jax import lax
from jax.experimental import pallas as pl
from jax.experimental.pallas import tpu as pltpu
from jax.experimental.pallas import tpu_sc as plsc

B = 4096
D = 256
K = 8192
TEMP = 0.05
BR = 512
COLT = 2048
NBLK = B // BR


def _sc_gather(targets, feats):
    info = plsc.get_sparse_core_info()
    nw = info.num_cores * info.num_subcores
    bpw = B // nw
    mesh = plsc.VectorSubcoreMesh(core_axis_name="c", subcore_axis_name="s")

    @functools.partial(
        pl.kernel, mesh=mesh,
        out_type=(jax.ShapeDtypeStruct((B, D), jnp.float32),
                  jax.ShapeDtypeStruct((B, D), jnp.float32)),
        scratch_types=[
            pltpu.VMEM((bpw,), jnp.int32),
            pltpu.VMEM((bpw,), jnp.int32),
            pltpu.VMEM((bpw, D), jnp.float32),
            pltpu.VMEM((bpw, D), jnp.float32),
            pltpu.SemaphoreType.DMA,
        ],
    )
    def k(tgt_hbm, feats_hbm, outm_hbm, outh_hbm, idx_v, idx2_v,
          rows_m, rows_h, sem):
        wid = lax.axis_index("s") * info.num_cores + lax.axis_index("c")
        base = wid * bpw
        pltpu.sync_copy(tgt_hbm.at[pl.ds(base, bpw)], idx_v)
        for j in range(bpw // 16):
            sl = pl.ds(j * 16, 16)
            idx2_v[sl] = idx_v[sl] + K
        pltpu.async_copy(feats_hbm.at[idx_v], rows_m, sem).wait()
        pltpu.async_copy(feats_hbm.at[idx2_v], rows_h, sem).wait()
        pltpu.sync_copy(rows_m, outm_hbm.at[pl.ds(base, bpw)])
        pltpu.sync_copy(rows_h, outh_hbm.at[pl.ds(base, bpw)])

    return k(targets, feats)


def _tc_body(x_ref, gm_ref, gh_ref, feats_ref, out_ref, fb_ref):
    i = pl.program_id(0)

    @pl.when(i == 0)
    def _cast():
        fb_ref[...] = feats_ref[...].astype(jnp.bfloat16)

    x = x_ref[...]
    norm = jnp.sqrt(jnp.sum(x * x, axis=1, keepdims=True))
    xn = x / jnp.maximum(norm, 1e-12)
    t_m = jnp.sum(xn * gm_ref[...], axis=1) * (1.0 / TEMP)
    t_h = jnp.sum(xn * gh_ref[...], axis=1) * (1.0 / TEMP)
    LOG2E = 1.4426950408889634
    xnb = (xn * (LOG2E / TEMP)).astype(jnp.bfloat16)
    acc_m = jnp.zeros((BR,), jnp.float32)
    acc_h = jnp.zeros((BR,), jnp.float32)
    for c in range(K // COLT):
        f_m = fb_ref[pl.ds(c * COLT, COLT), :]
        l_m = lax.dot_general(xnb, f_m, (((1,), (1,)), ((), ())),
                              preferred_element_type=jnp.float32)
        acc_m = acc_m + jnp.sum(jnp.exp2(l_m), axis=1)
        f_h = fb_ref[pl.ds(K + c * COLT, COLT), :]
        l_h = lax.dot_general(xnb, f_h, (((1,), (1,)), ((), ())),
                              preferred_element_type=jnp.float32)
        acc_h = acc_h + jnp.sum(jnp.exp2(l_h), axis=1)
    lse_m = jnp.log(acc_m)
    lse_h = jnp.log(acc_h)
    block = jnp.sum((lse_m - t_m) + (lse_h - t_h))

    @pl.when(i == 0)
    def _init():
        out_ref[0, 0] = 0.0

    out_ref[0, 0] += block

    @pl.when(i == NBLK - 1)
    def _fin():
        out_ref[0, 0] = out_ref[0, 0] * (0.5 / B)


def _tc_call(x, g_m, g_h, feats, interpret=False):
    return pl.pallas_call(
        _tc_body,
        grid=(NBLK,),
        in_specs=[
            pl.BlockSpec((BR, D), lambda i: (i, 0)),
            pl.BlockSpec((BR, D), lambda i: (i, 0)),
            pl.BlockSpec((BR, D), lambda i: (i, 0)),
            pl.BlockSpec((2 * K, D), lambda i: (0, 0)),
        ],
        out_specs=pl.BlockSpec((1, 1), lambda i: (0, 0),
                               memory_space=pltpu.SMEM),
        out_shape=jax.ShapeDtypeStruct((1, 1), jnp.float32),
        scratch_shapes=[pltpu.VMEM((2 * K, D), jnp.bfloat16)],
        interpret=interpret,
    )(x, g_m, g_h, feats)


def kernel(inputs, targets, features):
    tgt = targets.astype(jnp.int32)
    g_m, g_h = _sc_gather(tgt, features)
    out = _tc_call(inputs, g_m, g_h, features)
    return out[0, 0]


# speedup vs baseline: 6.2508x; 1.0524x over previous
import functools

import jax
import jax.numpy as jnp
from jax import lax
from jax.experimental import pallas as pl
from jax.experimental.pallas import tpu as pltpu
from jax.experimental.pallas import tpu_sc as plsc

B = 4096
D = 256
K = 8192
TEMP = 0.05
BR = 512
COLT = 2048
NBLK = B // BR
LOG2E = 1.4426950408889634


def _sc_gather(targets, feats):
    info = plsc.get_sparse_core_info()
    nw = info.num_cores * info.num_subcores
    bpw = B // nw
    mesh = plsc.VectorSubcoreMesh(core_axis_name="c", subcore_axis_name="s")

    @functools.partial(
        pl.kernel, mesh=mesh,
        out_type=(jax.ShapeDtypeStruct((B, D), jnp.float32),
                  jax.ShapeDtypeStruct((B, D), jnp.float32)),
        scratch_types=[
            pltpu.VMEM((bpw,), jnp.int32),
            pltpu.VMEM((bpw,), jnp.int32),
            pltpu.VMEM((bpw, D), jnp.float32),
            pltpu.VMEM((bpw, D), jnp.float32),
            pltpu.SemaphoreType.DMA,
        ],
    )
    def k(tgt_hbm, feats_hbm, outm_hbm, outh_hbm, idx_v, idx2_v,
          rows_m, rows_h, sem):
        wid = lax.axis_index("s") * info.num_cores + lax.axis_index("c")
        base = wid * bpw
        pltpu.sync_copy(tgt_hbm.at[pl.ds(base, bpw)], idx_v)
        for j in range(bpw // 16):
            sl = pl.ds(j * 16, 16)
            idx2_v[sl] = idx_v[sl] + K
        pltpu.async_copy(feats_hbm.at[idx_v], rows_m, sem).wait()
        pltpu.async_copy(feats_hbm.at[idx2_v], rows_h, sem).wait()
        pltpu.sync_copy(rows_m, outm_hbm.at[pl.ds(base, bpw)])
        pltpu.sync_copy(rows_h, outh_hbm.at[pl.ds(base, bpw)])

    return k(targets, feats)


def _lse_body(x_ref, feats_ref, lse_ref, fb_ref):
    i = pl.program_id(0)

    @pl.when(i == 0)
    def _cast():
        fb_ref[...] = feats_ref[...].astype(jnp.bfloat16)

    x = x_ref[...]
    norm = jnp.sqrt(jnp.sum(x * x, axis=1, keepdims=True))
    xn = x / jnp.maximum(norm, 1e-12)
    xnb = (xn * (LOG2E / TEMP)).astype(jnp.bfloat16)
    acc_m = jnp.zeros((BR,), jnp.float32)
    acc_h = jnp.zeros((BR,), jnp.float32)
    for c in range(K // COLT):
        f_m = fb_ref[pl.ds(c * COLT, COLT), :]
        l_m = lax.dot_general(xnb, f_m, (((1,), (1,)), ((), ())),
                              preferred_element_type=jnp.float32)
        acc_m = acc_m + jnp.sum(jnp.exp2(l_m), axis=1)
        f_h = fb_ref[pl.ds(K + c * COLT, COLT), :]
        l_h = lax.dot_general(xnb, f_h, (((1,), (1,)), ((), ())),
                              preferred_element_type=jnp.float32)
        acc_h = acc_h + jnp.sum(jnp.exp2(l_h), axis=1)
    lse_ref[:, 0] = jnp.log(acc_m)
    lse_ref[:, 1] = jnp.log(acc_h)


def _lse_call(x, feats, interpret=False):
    return pl.pallas_call(
        _lse_body,
        grid=(NBLK,),
        in_specs=[
            pl.BlockSpec((BR, D), lambda i: (i, 0)),
            pl.BlockSpec((2 * K, D), lambda i: (0, 0)),
        ],
        out_specs=pl.BlockSpec((BR, 2), lambda i: (i, 0)),
        out_shape=jax.ShapeDtypeStruct((B, 2), jnp.float32),
        scratch_shapes=[pltpu.VMEM((2 * K, D), jnp.bfloat16)],
        interpret=interpret,
    )(x, feats)


def _combine_body(x_ref, gm_ref, gh_ref, lse_ref, out_ref):
    x = x_ref[...]
    norm = jnp.sqrt(jnp.sum(x * x, axis=1, keepdims=True))
    xn = x / jnp.maximum(norm, 1e-12)
    t_m = jnp.sum(xn * gm_ref[...], axis=1) * (1.0 / TEMP)
    t_h = jnp.sum(xn * gh_ref[...], axis=1) * (1.0 / TEMP)
    total = jnp.sum((lse_ref[:, 0] - t_m) + (lse_ref[:, 1] - t_h))
    out_ref[0, 0] = total * (0.5 / B)


def _combine_call(x, g_m, g_h, lse, interpret=False):
    return pl.pallas_call(
        _combine_body,
        out_specs=pl.BlockSpec(memory_space=pltpu.SMEM),
        out_shape=jax.ShapeDtypeStruct((1, 1), jnp.float32),
        interpret=interpret,
    )(x, g_m, g_h, lse)


def kernel(inputs, targets, features):
    tgt = targets.astype(jnp.int32)
    g_m, g_h = _sc_gather(tgt, features)
    lse = _lse_call(inputs, features)
    out = _combine_call(inputs, g_m, g_h, lse)
    return out[0, 0]
